# baseline (speedup 1.0000x reference)
import functools

import jax
import jax.numpy as jnp
from jax import lax
from jax.experimental import pallas as pl
from jax.experimental.pallas import tpu as pltpu
from jax.experimental.pallas import tpu_sc as plsc

WD = 64
AD = 32
NNEG = 5
NR = 4096
L = 200
BATCH = 1024
NNZ = 16384
AVG_RATING = 3.8

_NC = 2
_NS = 16
_NW = _NC * _NS


def _sc_gather(table, idx, chunk):
    n = idx.shape[0]
    d = table.shape[1]
    b_per_w = n // _NW
    nchunks = b_per_w // chunk
    mesh = plsc.VectorSubcoreMesh(core_axis_name="c", subcore_axis_name="s")

    @functools.partial(
        pl.kernel,
        mesh=mesh,
        compiler_params=pltpu.CompilerParams(use_tc_tiling_on_sc=False),
        out_type=jax.ShapeDtypeStruct((n, d), table.dtype),
        scratch_types=[
            pltpu.VMEM((chunk,), jnp.int32),
            pltpu.VMEM((chunk, d), table.dtype),
            pltpu.SemaphoreType.DMA,
        ],
    )
    def k(table_hbm, idx_hbm, out_hbm, idx_v, rows_v, sem):
        wid = lax.axis_index("s") * _NC + lax.axis_index("c")

        @pl.loop(0, nchunks)
        def _(c):
            base = wid * b_per_w + c * chunk
            pltpu.sync_copy(idx_hbm.at[pl.ds(base, chunk)], idx_v)
            pltpu.async_copy(table_hbm.at[idx_v], rows_v, sem).wait()
            pltpu.sync_copy(rows_v, out_hbm.at[pl.ds(base, chunk)])

    return k(table, idx)


def _l2n(x):
    n = jnp.sqrt(jnp.sum(x * x, axis=1, keepdims=True))
    return x / jnp.maximum(n, 1e-12)


def _abae_block(e_ref, rp_ref, rn_ref, mw_ref, wwt_ref, wb_ref,
                twt_ref, rs_ref, ab_ref):
    br = e_ref.shape[0]
    e2d = e_ref[...]
    e3 = e2d.reshape(br, L * WD // 128, 128)
    q = jnp.dot(rp_ref[...], mw_ref[...], preferred_element_type=jnp.float32)
    qq = jnp.concatenate([q, q], axis=1)
    w1 = e3 * qq[:, None, :]
    lo = jnp.sum(w1[:, :, :WD], axis=2)
    hi = jnp.sum(w1[:, :, WD:], axis=2)
    dx = jnp.stack([lo, hi], axis=2).reshape(br, L)
    ex = jnp.exp(dx)
    sumex = jnp.sum(ex, axis=1, keepdims=True)
    per = jnp.concatenate([ex] * 16, axis=1)
    exf = jnp.concatenate([per] * 4, axis=1)
    w2 = e2d * exf
    zcols = [jnp.sum(w2[:, i * L:(i + 1) * L], axis=1, keepdims=True)
             for i in range(WD)]
    z_s = jnp.concatenate(zcols, axis=1) / sumex
    logits = jnp.dot(z_s, wwt_ref[...], preferred_element_type=jnp.float32)
    logits = logits + wb_ref[...]
    le = jnp.exp(logits)
    p_t = le / jnp.sum(le, axis=1, keepdims=True)
    r_s = jnp.dot(p_t, twt_ref[...], preferred_element_type=jnp.float32)
    rs_ref[...] = r_s
    rsn = _l2n(r_s)
    zsn = _l2n(z_s)
    c1 = jnp.sum(rsn * zsn, axis=1, keepdims=True)
    c2s = []
    for nn in range(NNEG):
        znn = _l2n(rn_ref[:, nn * WD:(nn + 1) * WD])
        c2s.append(jnp.sum(znn * rsn, axis=1, keepdims=True))
    c2 = jnp.concatenate(c2s, axis=1)
    ab_ref[...] = jnp.maximum(0.0, 1.0 - (c1 - c2))


def _abae_call(e_flat2d, rp, rn2, m_w, w_wt, w_b2, t_wt, br):
    nr = e_flat2d.shape[0]
    grid = (nr // br,)
    return pl.pallas_call(
        _abae_block,
        grid=grid,
        in_specs=[
            pl.BlockSpec((br, L * WD), lambda i: (i, 0)),
            pl.BlockSpec((br, WD), lambda i: (i, 0)),
            pl.BlockSpec((br, NNEG * WD), lambda i: (i, 0)),
            pl.BlockSpec((WD, WD), lambda i: (0, 0)),
            pl.BlockSpec((WD, AD), lambda i: (0, 0)),
            pl.BlockSpec((1, AD), lambda i: (0, 0)),
            pl.BlockSpec((AD, WD), lambda i: (0, 0)),
        ],
        out_specs=[
            pl.BlockSpec((br, WD), lambda i: (i, 0)),
            pl.BlockSpec((br, NNEG), lambda i: (i, 0)),
        ],
        out_shape=[
            jax.ShapeDtypeStruct((nr, WD), jnp.float32),
            jax.ShapeDtypeStruct((nr, NNEG), jnp.float32),
        ],
    )(e_flat2d, rp, rn2, m_w, w_wt, w_b2, t_wt)


def _rating_block(g_ref, vu_ref, vi_ref, iu_ref, ii_ref, lab_ref, tw_ref,
                  twt_ref, ab_ref, obj_ref, rl_ref, pred_ref):
    ch = 2048
    rows = lax.broadcasted_iota(jnp.int32, (BATCH, ch), 0)
    accu = jnp.zeros((BATCH, WD), jnp.float32)
    acci = jnp.zeros((BATCH, WD), jnp.float32)
    for c in range(NNZ // ch):
        sl = pl.ds(c * ch, ch)
        pu = (rows == iu_ref[:, sl]).astype(jnp.float32) * vu_ref[:, sl]
        accu = accu + jnp.dot(pu, g_ref[pl.ds(c * ch, ch), :],
                              preferred_element_type=jnp.float32)
        pi = (rows == ii_ref[:, sl]).astype(jnp.float32) * vi_ref[:, sl]
        acci = acci + jnp.dot(pi, g_ref[pl.ds(NNZ + c * ch, ch), :],
                              preferred_element_type=jnp.float32)
    pred = jnp.sum(accu * acci, axis=1) + AVG_RATING
    rl = (pred - lab_ref[0, :]) ** 2
    rl_ref[0, :] = rl
    pred_ref[...] = pred[:, None]
    cs = jnp.sum(tw_ref[...] * tw_ref[...], axis=0, keepdims=True)
    inv = 1.0 / jnp.maximum(jnp.sqrt(cs), 1e-12)
    tnt = twt_ref[...] * inv[0, :, None]
    tn = tw_ref[...] * inv
    tt = jnp.dot(tnt, tn, preferred_element_type=jnp.float32)
    eye = (lax.broadcasted_iota(jnp.int32, (AD, AD), 0) ==
           lax.broadcasted_iota(jnp.int32, (AD, AD), 1)).astype(jnp.float32)
    u_loss = jnp.sum((tt - eye) ** 2) / (AD * AD)
    j_loss = jnp.sum(ab_ref[...]) / (NR * NNEG)
    obj = jnp.sum(rl) / BATCH + u_loss + j_loss
    obj_ref[...] = jnp.broadcast_to(obj, (1, 1))


def _rating_call(g, vu, vi, iu, ii, lab, t_w, t_wt, ab):
    return pl.pallas_call(
        _rating_block,
        out_shape=[
            jax.ShapeDtypeStruct((1, 1), jnp.float32),
            jax.ShapeDtypeStruct((1, BATCH), jnp.float32),
            jax.ShapeDtypeStruct((BATCH, 1), jnp.float32),
        ],
    )(g, vu, vi, iu, ii, lab, t_w, t_wt, ab)


def kernel(historical_review, review_positive, review_negative, user, item,
           label, user_histor_index, user_histor_value, item_histor_index,
           item_histor_value, emb_table, M_w, W_w, W_b, T_w):
    hr = historical_review.reshape(-1).astype(jnp.int32)
    rn2 = review_negative.reshape(NR, NNEG * WD)
    n_slices = 1
    sr = NR // n_slices
    rs_parts, ab_parts = [], []
    for s in range(n_slices):
        hr_s = lax.dynamic_slice_in_dim(hr, s * sr * L, sr * L)
        e_flat = _sc_gather(emb_table, hr_s, 640)
        rs_s, ab_s = _abae_call(
            e_flat.reshape(sr, L * WD),
            lax.dynamic_slice_in_dim(review_positive, s * sr, sr),
            lax.dynamic_slice_in_dim(rn2, s * sr, sr),
            M_w, W_w.T, W_b.reshape(1, AD), T_w.T, 128)
        rs_parts.append(rs_s)
        ab_parts.append(ab_s)
    r_s = jnp.concatenate(rs_parts, axis=0)
    abae = jnp.concatenate(ab_parts, axis=0)
    idx_cat = jnp.concatenate(
        [user_histor_index[1], item_histor_index[1]]).astype(jnp.int32)
    g = _sc_gather(r_s, idx_cat, 1024)
    obj, rl, pred = _rating_call(
        g,
        user_histor_value.reshape(1, NNZ),
        item_histor_value.reshape(1, NNZ),
        user_histor_index[0].reshape(1, NNZ).astype(jnp.int32),
        item_histor_index[0].reshape(1, NNZ).astype(jnp.int32),
        label.reshape(1, BATCH),
        T_w,
        T_w.T,
        abae,
    )
    return (obj.reshape(()), rl.reshape(-1), abae.reshape(-1), pred)


# speedup vs baseline: 1.0814x; 1.0814x over previous
import functools

import jax
import jax.numpy as jnp
from jax import lax
from jax.experimental import pallas as pl
from jax.experimental.pallas import tpu as pltpu
from jax.experimental.pallas import tpu_sc as plsc

WD = 64
AD = 32
NNEG = 5
NR = 4096
L = 200
BATCH = 1024
NNZ = 16384
AVG_RATING = 3.8

_NC = 2
_NS = 16
_NW = _NC * _NS


def _sc_gather(table, idx, chunk):
    n = idx.shape[0]
    d = table.shape[1]
    b_per_w = n // _NW
    nchunks = b_per_w // chunk
    mesh = plsc.VectorSubcoreMesh(core_axis_name="c", subcore_axis_name="s")

    @functools.partial(
        pl.kernel,
        mesh=mesh,
        compiler_params=pltpu.CompilerParams(use_tc_tiling_on_sc=False),
        out_type=jax.ShapeDtypeStruct((n, d), table.dtype),
        scratch_types=[
            pltpu.VMEM((chunk,), jnp.int32),
            pltpu.VMEM((chunk, d), table.dtype),
            pltpu.SemaphoreType.DMA,
        ],
    )
    def k(table_hbm, idx_hbm, out_hbm, idx_v, rows_v, sem):
        wid = lax.axis_index("s") * _NC + lax.axis_index("c")

        @pl.loop(0, nchunks)
        def _(c):
            base = wid * b_per_w + c * chunk
            pltpu.sync_copy(idx_hbm.at[pl.ds(base, chunk)], idx_v)
            pltpu.async_copy(table_hbm.at[idx_v], rows_v, sem).wait()
            pltpu.sync_copy(rows_v, out_hbm.at[pl.ds(base, chunk)])

    return k(table, idx)


def _l2n(x):
    n = jnp.sqrt(jnp.sum(x * x, axis=1, keepdims=True))
    return x / jnp.maximum(n, 1e-12)


def _abae_block(e_ref, rp_ref, rn_ref, mw_ref, wwt_ref, wb_ref,
                twt_ref, rs_ref, ab_ref):
    br = e_ref.shape[0]
    e2d = e_ref[...]
    e3 = e2d.reshape(br, L * WD // 128, 128)
    q = jnp.dot(rp_ref[...], mw_ref[...], preferred_element_type=jnp.float32)
    qq = jnp.concatenate([q, q], axis=1)
    w1 = e3 * qq[:, None, :]
    lo = jnp.sum(w1[:, :, :WD], axis=2)
    hi = jnp.sum(w1[:, :, WD:], axis=2)
    nt = L * WD // 128
    ti = lax.broadcasted_iota(jnp.int32, (nt, L), 0)
    ci = lax.broadcasted_iota(jnp.int32, (nt, L), 1)
    pe = (ci == 2 * ti).astype(jnp.float32)
    po = (ci == 2 * ti + 1).astype(jnp.float32)
    dx = (jnp.dot(lo, pe, preferred_element_type=jnp.float32) +
          jnp.dot(hi, po, preferred_element_type=jnp.float32))
    ex = jnp.exp(dx)
    sumex = jnp.sum(ex, axis=1, keepdims=True)
    per = jnp.concatenate([ex] * 16, axis=1)
    exf = jnp.concatenate([per] * 4, axis=1)
    w2 = e2d * exf
    zcols = [jnp.sum(w2[:, i * L:(i + 1) * L], axis=1, keepdims=True)
             for i in range(WD)]
    z_s = jnp.concatenate(zcols, axis=1) / sumex
    logits = jnp.dot(z_s, wwt_ref[...], preferred_element_type=jnp.float32)
    logits = logits + wb_ref[...]
    le = jnp.exp(logits)
    p_t = le / jnp.sum(le, axis=1, keepdims=True)
    r_s = jnp.dot(p_t, twt_ref[...], preferred_element_type=jnp.float32)
    rs_ref[...] = r_s
    rsn = _l2n(r_s)
    zsn = _l2n(z_s)
    c1 = jnp.sum(rsn * zsn, axis=1, keepdims=True)
    c2s = []
    for nn in range(NNEG):
        znn = _l2n(rn_ref[:, nn * WD:(nn + 1) * WD])
        c2s.append(jnp.sum(znn * rsn, axis=1, keepdims=True))
    c2 = jnp.concatenate(c2s, axis=1)
    ab_ref[...] = jnp.maximum(0.0, 1.0 - (c1 - c2))


def _abae_call(e_flat2d, rp, rn2, m_w, w_wt, w_b2, t_wt, br):
    nr = e_flat2d.shape[0]
    grid = (nr // br,)
    return pl.pallas_call(
        _abae_block,
        grid=grid,
        in_specs=[
            pl.BlockSpec((br, L * WD), lambda i: (i, 0)),
            pl.BlockSpec((br, WD), lambda i: (i, 0)),
            pl.BlockSpec((br, NNEG * WD), lambda i: (i, 0)),
            pl.BlockSpec((WD, WD), lambda i: (0, 0)),
            pl.BlockSpec((WD, AD), lambda i: (0, 0)),
            pl.BlockSpec((1, AD), lambda i: (0, 0)),
            pl.BlockSpec((AD, WD), lambda i: (0, 0)),
        ],
        out_specs=[
            pl.BlockSpec((br, WD), lambda i: (i, 0)),
            pl.BlockSpec((br, NNEG), lambda i: (i, 0)),
        ],
        out_shape=[
            jax.ShapeDtypeStruct((nr, WD), jnp.float32),
            jax.ShapeDtypeStruct((nr, NNEG), jnp.float32),
        ],
    )(e_flat2d, rp, rn2, m_w, w_wt, w_b2, t_wt)


def _rating_block(g_ref, vu_ref, vi_ref, iu_ref, ii_ref, lab_ref, tw_ref,
                  twt_ref, ab_ref, obj_ref, rl_ref, pred_ref):
    ch = 2048
    rows = lax.broadcasted_iota(jnp.int32, (BATCH, ch), 0)
    accu = jnp.zeros((BATCH, WD), jnp.float32)
    acci = jnp.zeros((BATCH, WD), jnp.float32)
    for c in range(NNZ // ch):
        sl = pl.ds(c * ch, ch)
        pu = (rows == iu_ref[:, sl]).astype(jnp.float32) * vu_ref[:, sl]
        accu = accu + jnp.dot(pu, g_ref[pl.ds(c * ch, ch), :],
                              preferred_element_type=jnp.float32)
        pi = (rows == ii_ref[:, sl]).astype(jnp.float32) * vi_ref[:, sl]
        acci = acci + jnp.dot(pi, g_ref[pl.ds(NNZ + c * ch, ch), :],
                              preferred_element_type=jnp.float32)
    pred = jnp.sum(accu * acci, axis=1) + AVG_RATING
    rl = (pred - lab_ref[0, :]) ** 2
    rl_ref[0, :] = rl
    pred_ref[...] = pred[:, None]
    cs = jnp.sum(tw_ref[...] * tw_ref[...], axis=0, keepdims=True)
    inv = 1.0 / jnp.maximum(jnp.sqrt(cs), 1e-12)
    tnt = twt_ref[...] * inv[0, :, None]
    tn = tw_ref[...] * inv
    tt = jnp.dot(tnt, tn, preferred_element_type=jnp.float32)
    eye = (lax.broadcasted_iota(jnp.int32, (AD, AD), 0) ==
           lax.broadcasted_iota(jnp.int32, (AD, AD), 1)).astype(jnp.float32)
    u_loss = jnp.sum((tt - eye) ** 2) / (AD * AD)
    j_loss = jnp.sum(ab_ref[...]) / (NR * NNEG)
    obj = jnp.sum(rl) / BATCH + u_loss + j_loss
    obj_ref[...] = jnp.broadcast_to(obj, (1, 1))


def _rating_call(g, vu, vi, iu, ii, lab, t_w, t_wt, ab):
    return pl.pallas_call(
        _rating_block,
        out_shape=[
            jax.ShapeDtypeStruct((1, 1), jnp.float32),
            jax.ShapeDtypeStruct((1, BATCH), jnp.float32),
            jax.ShapeDtypeStruct((BATCH, 1), jnp.float32),
        ],
    )(g, vu, vi, iu, ii, lab, t_w, t_wt, ab)


def kernel(historical_review, review_positive, review_negative, user, item,
           label, user_histor_index, user_histor_value, item_histor_index,
           item_histor_value, emb_table, M_w, W_w, W_b, T_w):
    hr = historical_review.reshape(-1).astype(jnp.int32)
    rn2 = review_negative.reshape(NR, NNEG * WD)
    n_slices = 4
    sr = NR // n_slices
    rs_parts, ab_parts = [], []
    for s in range(n_slices):
        hr_s = lax.dynamic_slice_in_dim(hr, s * sr * L, sr * L)
        e_flat = _sc_gather(emb_table, hr_s, 640)
        rs_s, ab_s = _abae_call(
            e_flat.reshape(sr, L * WD),
            lax.dynamic_slice_in_dim(review_positive, s * sr, sr),
            lax.dynamic_slice_in_dim(rn2, s * sr, sr),
            M_w, W_w.T, W_b.reshape(1, AD), T_w.T, 128)
        rs_parts.append(rs_s)
        ab_parts.append(ab_s)
    r_s = jnp.concatenate(rs_parts, axis=0)
    abae = jnp.concatenate(ab_parts, axis=0)
    idx_cat = jnp.concatenate(
        [user_histor_index[1], item_histor_index[1]]).astype(jnp.int32)
    g = _sc_gather(r_s, idx_cat, 1024)
    obj, rl, pred = _rating_call(
        g,
        user_histor_value.reshape(1, NNZ),
        item_histor_value.reshape(1, NNZ),
        user_histor_index[0].reshape(1, NNZ).astype(jnp.int32),
        item_histor_index[0].reshape(1, NNZ).astype(jnp.int32),
        label.reshape(1, BATCH),
        T_w,
        T_w.T,
        abae,
    )
    return (obj.reshape(()), rl.reshape(-1), abae.reshape(-1), pred)
